# baseline (speedup 1.0000x reference)
import functools
import jax
import jax.numpy as jnp
from jax import lax
from jax.experimental import pallas as pl
from jax.experimental.pallas import tpu as pltpu
from jax.experimental.pallas import tpu_sc as plsc

N = 10000
E = 320000
F = 128
OUT = 2048
L = 6
G = 64

NP = 10240
NB = 64
NPB = 160
NWC = 16
NCORE = 2
EPW = E // (NWC * NCORE)
CHA = 2000
NCHA = EPW // CHA
CHB = 192
CHS = 512
WSORT = 2048
OFFR = 168
CAPC = E // 2 + NB * NWC * 8 + CHS
MAGIC = 52429
FBIG = 3.0e38


def _wid():
    return lax.axis_index("s") * NCORE + lax.axis_index("c")


def _sc_partition_body(esrc_hbm, edst_hbm, edges_hbm, srt_hbm, bmeta_hbm,
                       offtab_hbm,
                       src_v, dst_v, stage_v, pub_v, rb_v, meta_v, win_v,
                       shared_v, cnt_s, base_s, fill_s, gst_s, gln_s,
                       cnt2_s, off_s):
    c = lax.axis_index("c")
    s = lax.axis_index("s")
    ebase = (c * NWC + s) * EPW

    def init_cnt(b, _):
        cnt_s[b] = 0
        return 0
    lax.fori_loop(0, NB, init_cnt, 0)

    def count_chunk(ch, _):
        pltpu.sync_copy(edst_hbm.at[pl.ds(ebase + ch * CHA, CHA)], dst_v)
        def grp(g, _):
            d16 = dst_v[pl.ds(g * 16, 16)]
            b16 = lax.shift_right_logical(d16 * MAGIC, 23)
            for j in range(16):
                b = b16[j]
                cnt_s[b] = cnt_s[b] + 1
            return 0
        lax.fori_loop(0, CHA // 16, grp, 0)
        return 0
    lax.fori_loop(0, NCHA, count_chunk, 0)

    def pub(b, _):
        pub_v[b, :] = jnp.full((16,), cnt_s[b], jnp.int32)
        return 0
    lax.fori_loop(0, NB, pub, 0)
    pltpu.sync_copy(pub_v, shared_v.at[s])
    plsc.subcore_barrier()
    pltpu.sync_copy(shared_v, rb_v)

    def bucket_base(b, run):
        def acc_w(w, carry):
            part, tot = carry
            cnt = rb_v[w, b, :][0]
            r8 = (cnt + 7) & (-8)
            part = part + jnp.where(w < s, r8, 0)
            return (part, tot + r8)
        part, tot = lax.fori_loop(0, NWC, acc_w, (0, 0))
        base_s[b] = run + part
        fill_s[b] = 0
        gst_s[b] = run
        gln_s[b] = tot
        meta_v[b, :] = jnp.full((16,), run, jnp.int32)
        meta_v[NB + b, :] = jnp.full((16,), tot, jnp.int32)
        return run + tot
    lax.fori_loop(0, NB, bucket_base, 0)

    @pl.when(s == 0)
    def _():
        mrow = pl.multiple_of(c * 2 * NB, 8)
        pltpu.sync_copy(meta_v, bmeta_hbm.at[pl.ds(mrow, 2 * NB)])

    crow = c * CAPC

    def place_chunk(ch, _):
        pltpu.sync_copy(esrc_hbm.at[pl.ds(ebase + ch * CHA, CHA)], src_v)
        pltpu.sync_copy(edst_hbm.at[pl.ds(ebase + ch * CHA, CHA)], dst_v)
        def grp(g, _):
            s16 = src_v[pl.ds(g * 16, 16)]
            d16 = dst_v[pl.ds(g * 16, 16)]
            b16 = lax.shift_right_logical(d16 * MAGIC, 23)
            p16 = s16 | lax.shift_left(d16, 14)
            for j in range(16):
                b = b16[j]
                pk = p16[j]
                f = fill_s[b]
                stage_v[b * 8 + (f & 7), :] = jnp.full((16,), pk, jnp.int32)
                fill_s[b] = f + 1
                @pl.when((f & 7) == 7)
                def _():
                    row0 = pl.multiple_of(crow + base_s[b] + f - 7, 8)
                    pltpu.sync_copy(stage_v.at[pl.ds(b * 8, 8)],
                                    edges_hbm.at[pl.ds(row0, 8)])
            return 0
        lax.fori_loop(0, CHA // 16, grp, 0)
        return 0
    lax.fori_loop(0, NCHA, place_chunk, 0)

    def tail(b, _):
        f = fill_s[b]
        rem = f & 7
        @pl.when(rem > 0)
        def _():
            def padrow(r, _):
                @pl.when(r >= rem)
                def _():
                    stage_v[b * 8 + r, :] = jnp.full((16,), -1, jnp.int32)
                return 0
            lax.fori_loop(0, 8, padrow, 0)
            row0 = pl.multiple_of(crow + base_s[b] + f - rem, 8)
            pltpu.sync_copy(stage_v.at[pl.ds(b * 8, 8)],
                            edges_hbm.at[pl.ds(row0, 8)])
        return 0
    lax.fori_loop(0, NB, tail, 0)

    plsc.subcore_barrier()
    iota16 = lax.iota(jnp.int32, 16)

    def sort_bucket(bb, _):
        b = s * 4 + bb
        lo = b * NPB
        gstart = gst_s[b]
        glen = gln_s[b]
        nch = (glen + CHS - 1) // CHS

        def zcnt(i, _):
            cnt2_s[i] = 0
            return 0
        lax.fori_loop(0, NPB, zcnt, 0)

        def count_ch(ch, _):
            row0 = pl.multiple_of(crow + gstart + ch * CHS, 8)
            pltpu.sync_copy(edges_hbm.at[pl.ds(row0, CHS)], stage_v)
            nval = glen - ch * CHS
            def cgrp(g, _):
                pk16 = plsc.load_gather(
                    stage_v, [jnp.full((16,), g * 16, jnp.int32) + iota16,
                              jnp.zeros((16,), jnp.int32)])
                ld16 = lax.shift_right_arithmetic(pk16, 14) - lo
                rid = jnp.full((16,), g * 16, jnp.int32) + iota16
                ok16 = ((ld16 >= 0) & (ld16 < NPB)
                        & (rid < nval)).astype(jnp.int32)
                for j in range(16):
                    @pl.when(ok16[j] != 0)
                    def _():
                        ld = ld16[j]
                        cnt2_s[ld] = cnt2_s[ld] + 1
                return 0
            lax.fori_loop(0, CHS // 16, cgrp, 0)
            return 0
        lax.fori_loop(0, nch, count_ch, 0)

        def pref(i, run2):
            off_s[i] = run2
            return run2 + cnt2_s[i]
        vcnt = lax.fori_loop(0, NPB, pref, 0)

        def expoff(i, _):
            win_v[i, :] = jnp.full((16,), off_s[i], jnp.int32)
            return 0
        lax.fori_loop(0, NPB, expoff, 0)
        win_v[NPB, :] = jnp.full((16,), vcnt, jnp.int32)
        orow0 = pl.multiple_of((c * NB + b) * OFFR, 8)
        pltpu.sync_copy(win_v.at[pl.ds(0, OFFR)],
                        offtab_hbm.at[pl.ds(orow0, OFFR)])

        nwin = (glen + WSORT - 1) // WSORT

        def do_win(wnd, _):
            w0 = wnd * WSORT
            def rst(i, _):
                cnt2_s[i] = off_s[i]
                return 0
            lax.fori_loop(0, NPB, rst, 0)
            def prefill(i, _):
                win_v[i, :] = jnp.full((16,), -1, jnp.int32)
                return 0
            lax.fori_loop(0, WSORT, prefill, 0)

            def place_ch(ch, _):
                row0 = pl.multiple_of(crow + gstart + ch * CHS, 8)
                pltpu.sync_copy(edges_hbm.at[pl.ds(row0, CHS)], stage_v)
                nval = glen - ch * CHS
                def pgrp(g, _):
                    pk16 = plsc.load_gather(
                        stage_v, [jnp.full((16,), g * 16, jnp.int32) + iota16,
                                  jnp.zeros((16,), jnp.int32)])
                    ld16 = lax.shift_right_arithmetic(pk16, 14) - lo
                    rid = jnp.full((16,), g * 16, jnp.int32) + iota16
                    ok16 = ((ld16 >= 0) & (ld16 < NPB)
                            & (rid < nval)).astype(jnp.int32)
                    for j in range(16):
                        @pl.when(ok16[j] != 0)
                        def _():
                            ld = ld16[j]
                            pos = cnt2_s[ld]
                            cnt2_s[ld] = pos + 1
                            @pl.when((pos >= w0) & (pos < w0 + WSORT))
                            def _():
                                win_v[pos - w0, :] = jnp.full(
                                    (16,), pk16[j], jnp.int32)
                    return 0
                lax.fori_loop(0, CHS // 16, pgrp, 0)
                return 0
            lax.fori_loop(0, nch, place_ch, 0)

            nrows = jnp.minimum(WSORT, glen - w0)
            orow = crow + gstart + w0
            def w64(k, _):
                r0 = pl.multiple_of(orow + k * 64, 8)
                pltpu.sync_copy(win_v.at[pl.ds(k * 64, 64)],
                                srt_hbm.at[pl.ds(r0, 64)])
                return 0
            n64 = nrows // 64
            lax.fori_loop(0, n64, w64, 0)
            def w8(k, _):
                soff = pl.multiple_of(n64 * 64 + k * 8, 8)
                r0 = pl.multiple_of(orow + n64 * 64 + k * 8, 8)
                pltpu.sync_copy(win_v.at[pl.ds(soff, 8)],
                                srt_hbm.at[pl.ds(r0, 8)])
                return 0
            lax.fori_loop(0, (nrows - n64 * 64) // 8, w8, 0)
            return 0
        lax.fori_loop(0, nwin, do_win, 0)
        return 0
    lax.fori_loop(0, 4, sort_bucket, 0)


_sc_partition = functools.partial(
    pl.kernel,
    out_type=(
        jax.ShapeDtypeStruct((2 * CAPC, 16), jnp.int32),
        jax.ShapeDtypeStruct((2 * CAPC, 16), jnp.int32),
        jax.ShapeDtypeStruct((2 * 2 * NB, 16), jnp.int32),
        jax.ShapeDtypeStruct((2 * NB * OFFR, 16), jnp.int32),
    ),
    mesh=plsc.VectorSubcoreMesh(core_axis_name="c", subcore_axis_name="s"),
    scratch_types=[
        pltpu.VMEM((CHA,), jnp.int32),
        pltpu.VMEM((CHA,), jnp.int32),
        pltpu.VMEM((NB * 8, 16), jnp.int32),
        pltpu.VMEM((NB, 16), jnp.int32),
        pltpu.VMEM((NWC, NB, 16), jnp.int32),
        pltpu.VMEM((2 * NB, 16), jnp.int32),
        pltpu.VMEM((WSORT, 16), jnp.int32),
        pltpu.VMEM_SHARED((NWC, NB, 16), jnp.int32),
        pltpu.SMEM((NB,), jnp.int32),
        pltpu.SMEM((NB,), jnp.int32),
        pltpu.SMEM((NB,), jnp.int32),
        pltpu.SMEM((NB,), jnp.int32),
        pltpu.SMEM((NB,), jnp.int32),
        pltpu.SMEM((NPB,), jnp.int32),
        pltpu.SMEM((NPB,), jnp.int32),
    ],
    compiler_params=pltpu.CompilerParams(use_tc_tiling_on_sc=False, needs_layout_passes=False),
)(_sc_partition_body)


def _sc_aggregate_body(b_hbm, edges_hbm, bmeta_hbm, offtab_hbm,
                       s1_hbm, s2_hbm, mn_hbm, mx_hbm, dc_hbm,
                       acc_v, dc_v, pk_v, gidx_v, rows_v, meta_v, off_v, sem):
    c = lax.axis_index("c")
    s = lax.axis_index("s")
    w = s * NCORE + c
    iota16 = lax.iota(jnp.int32, 16)
    zf = jnp.zeros((16,), jnp.float32)
    bigf = jnp.full((16,), FBIG, jnp.float32)
    pltpu.sync_copy(bmeta_hbm, meta_v)

    id_regs = tuple([zf] * 8 + [zf] * 8 + [bigf] * 8 + [-bigf] * 8)

    def do_bucket(r, _):
        b = w * 2 + r
        lo = pl.multiple_of(b * NPB, 8)

        def initrow(i, _):
            for f in range(8):
                sl = pl.ds(f * 16, 16)
                acc_v[0, i, sl] = zf
                acc_v[1, i, sl] = zf
                acc_v[2, i, sl] = bigf
                acc_v[3, i, sl] = -bigf
            dc_v[i, :] = zf
            return 0
        lax.fori_loop(0, NPB, initrow, 0)

        for c2 in range(NCORE):
            gstart = meta_v[c2 * 2 * NB + b, :][0]
            crow = c2 * CAPC
            orow = pl.multiple_of((c2 * NB + b) * OFFR, 8)
            pltpu.sync_copy(offtab_hbm.at[pl.ds(orow, OFFR)], off_v)
            vcnt = off_v[NPB, :][0]

            def off_at(i):
                return off_v[i, :][0]

            def do_chunk(ch, node):
                c0 = ch * CHB
                cend = jnp.minimum(c0 + CHB, vcnt)
                row0 = pl.multiple_of(crow + gstart + c0, 8)
                pltpu.sync_copy(edges_hbm.at[pl.ds(row0, CHB)], pk_v)

                @plsc.parallel_loop(0, CHB // 16, unroll=4)
                def mkidx(g):
                    pkg = plsc.load_gather(
                        pk_v, [jnp.full((16,), g * 16, jnp.int32) + iota16,
                               jnp.zeros((16,), jnp.int32)])
                    srcg = jnp.minimum(pkg & 0x3FFF, NP - 1)
                    gidx_v[pl.ds(g * 16, 16)] = jnp.maximum(srcg, 0)
                pltpu.async_copy(b_hbm.at[gidx_v], rows_v, sem).wait()

                def wcond(carry):
                    node_, start_, cont_ = carry
                    return cont_ != 0

                def wbody(carry):
                    node_, start_, _ = carry
                    nend = off_at(node_ + 1)
                    seg0 = jnp.maximum(start_, c0)
                    seg1 = jnp.minimum(nend, cend)

                    def acc_edge(e, regs):
                        jr = e - c0
                        nr = []
                        for f in range(8):
                            rv = rows_v[jr, pl.ds(f * 16, 16)]
                            nr.append((regs[f] + rv,
                                       regs[8 + f] + rv * rv,
                                       jnp.minimum(regs[16 + f], rv),
                                       jnp.maximum(regs[24 + f], rv)))
                        return (tuple(t[0] for t in nr)
                                + tuple(t[1] for t in nr)
                                + tuple(t[2] for t in nr)
                                + tuple(t[3] for t in nr))
                    regs = plsc.parallel_loop(
                        seg0, seg1, unroll=4, carry=id_regs)(acc_edge)

                    @pl.when(seg1 > seg0)
                    def _():
                        for f in range(8):
                            sl = pl.ds(f * 16, 16)
                            plsc.addupdate(acc_v.at[0, node_, sl], regs[f])
                            plsc.addupdate(acc_v.at[1, node_, sl], regs[8 + f])
                            mnv = acc_v[2, node_, sl]
                            acc_v[2, node_, sl] = jnp.minimum(mnv, regs[16 + f])
                            mxv = acc_v[3, node_, sl]
                            acc_v[3, node_, sl] = jnp.maximum(mxv, regs[24 + f])
                        plsc.addupdate(
                            dc_v.at[node_],
                            zf + (seg1 - seg0).astype(jnp.float32))

                    adv = (nend <= cend).astype(jnp.int32)
                    node2 = jnp.where(adv != 0, node_ + 1, node_)
                    start2 = jnp.where(adv != 0, nend, start_)
                    cont2 = jnp.where(
                        (adv != 0) & (node2 < NPB) & (start2 < cend), 1, 0)
                    return (node2, start2, cont2)

                start0 = off_at(node)
                cont0 = jnp.where((node < NPB) & (start0 < cend), 1, 0)
                node, _, _ = lax.while_loop(wcond, wbody,
                                            (node, start0, cont0))
                return node

            nchunks = (vcnt + CHB - 1) // CHB
            lax.fori_loop(0, nchunks, do_chunk, jnp.int32(0))

        pltpu.sync_copy(acc_v.at[0], s1_hbm.at[pl.ds(lo, NPB)])
        pltpu.sync_copy(acc_v.at[1], s2_hbm.at[pl.ds(lo, NPB)])
        pltpu.sync_copy(acc_v.at[2], mn_hbm.at[pl.ds(lo, NPB)])
        pltpu.sync_copy(acc_v.at[3], mx_hbm.at[pl.ds(lo, NPB)])
        pltpu.sync_copy(dc_v, dc_hbm.at[pl.ds(lo, NPB)])
        return 0
    lax.fori_loop(0, 2, do_bucket, 0)


_sc_aggregate = functools.partial(
    pl.kernel,
    out_type=(
        jax.ShapeDtypeStruct((NP, F), jnp.float32),
        jax.ShapeDtypeStruct((NP, F), jnp.float32),
        jax.ShapeDtypeStruct((NP, F), jnp.float32),
        jax.ShapeDtypeStruct((NP, F), jnp.float32),
        jax.ShapeDtypeStruct((NP, 16), jnp.float32),
    ),
    mesh=plsc.VectorSubcoreMesh(core_axis_name="c", subcore_axis_name="s"),
    scratch_types=[
        pltpu.VMEM((4, NPB, F), jnp.float32),
        pltpu.VMEM((NPB, 16), jnp.float32),
        pltpu.VMEM((CHB, 16), jnp.int32),
        pltpu.VMEM((CHB,), jnp.int32),
        pltpu.VMEM((CHB, F), jnp.float32),
        pltpu.VMEM((2 * 2 * NB, 16), jnp.int32),
        pltpu.VMEM((OFFR, 16), jnp.int32),
        pltpu.SemaphoreType.DMA,
    ],
    compiler_params=pltpu.CompilerParams(use_tc_tiling_on_sc=False, needs_layout_passes=False),
)(_sc_aggregate_body)


def _k1_body(h_ref, w_ref, b_ref, a_out, b_out):
    h = h_ref[...]
    a_out[...] = jnp.dot(h, w_ref[0:F, :], preferred_element_type=jnp.float32)
    b_out[...] = jnp.dot(h, w_ref[F:2 * F, :],
                         preferred_element_type=jnp.float32) + b_ref[...]


def _k1(h, w, b):
    blk = 1024
    grid = NP // blk
    return pl.pallas_call(
        _k1_body,
        grid=(grid,),
        in_specs=[
            pl.BlockSpec((blk, F), lambda i: (i, 0)),
            pl.BlockSpec((2 * F, F), lambda i: (0, 0)),
            pl.BlockSpec((1, F), lambda i: (0, 0)),
        ],
        out_specs=[
            pl.BlockSpec((blk, F), lambda i: (i, 0)),
            pl.BlockSpec((blk, F), lambda i: (i, 0)),
        ],
        out_shape=[
            jax.ShapeDtypeStruct((NP, F), jnp.float32),
            jax.ShapeDtypeStruct((NP, F), jnp.float32),
        ],
    )(h, w, b.reshape(1, F))


def _k0_body(d_ref, out_ref):
    d = d_ref[...]
    out_ref[...] = (jnp.sum(jnp.log(d + 1.0)) / N).reshape(1, 1)


def _k0(d):
    return pl.pallas_call(
        _k0_body,
        out_shape=jax.ShapeDtypeStruct((1, 1), jnp.float32),
    )(d)


def _k2_body(h_ref, a_ref, s1_ref, s2_ref, mn_ref, mx_ref, d_ref, al_ref,
             pw_ref, pb_ref, lw_ref, lb_ref, out_ref, *, relu):
    h = h_ref[...]
    A = a_ref[...]
    S1 = s1_ref[...]
    S2 = s2_ref[...]
    d = d_ref[...]
    avg_log = al_ref[0, 0]
    deg_c = jnp.maximum(d, 1.0)
    has = (d > 0).astype(jnp.float32)
    mean = (d * A + S1) / deg_c
    mean_sq = (d * A * A + 2.0 * A * S1 + S2) / deg_c
    std = jnp.sqrt(jax.nn.relu(mean_sq - mean * mean) + 1e-5)
    mn = has * (A + jnp.where(d > 0, mn_ref[...], 0.0))
    mx = has * (A + jnp.where(d > 0, mx_ref[...], 0.0))
    lg = jnp.log(deg_c + 1.0)
    amp = lg / avg_log
    att = avg_log / lg
    parts = [h, mean, mn, mx, std,
             amp * mean, amp * mn, amp * mx, amp * std,
             att * mean, att * mn, att * mx, att * std]
    acc = pb_ref[...].astype(jnp.float32)
    o = jnp.zeros_like(h) + acc
    for i, p in enumerate(parts):
        o = o + jnp.dot(p, pw_ref[i * F:(i + 1) * F, :],
                        preferred_element_type=jnp.float32)
    o = jnp.dot(o, lw_ref[...], preferred_element_type=jnp.float32) + lb_ref[...]
    if relu:
        o = jax.nn.relu(o)
    out_ref[...] = o


def _k2(h, A, S1, S2, MN, MX, d, avg_log, pw, pb, lw, lb, relu):
    blk = 512
    grid = NP // blk
    body = functools.partial(_k2_body, relu=relu)
    return pl.pallas_call(
        body,
        grid=(grid,),
        in_specs=[
            pl.BlockSpec((blk, F), lambda i: (i, 0)),
            pl.BlockSpec((blk, F), lambda i: (i, 0)),
            pl.BlockSpec((blk, F), lambda i: (i, 0)),
            pl.BlockSpec((blk, F), lambda i: (i, 0)),
            pl.BlockSpec((blk, F), lambda i: (i, 0)),
            pl.BlockSpec((blk, F), lambda i: (i, 0)),
            pl.BlockSpec((blk, 1), lambda i: (i, 0)),
            pl.BlockSpec((1, 1), lambda i: (0, 0)),
            pl.BlockSpec((13 * F, F), lambda i: (0, 0)),
            pl.BlockSpec((1, F), lambda i: (0, 0)),
            pl.BlockSpec((F, F), lambda i: (0, 0)),
            pl.BlockSpec((1, F), lambda i: (0, 0)),
        ],
        out_specs=pl.BlockSpec((blk, F), lambda i: (i, 0)),
        out_shape=jax.ShapeDtypeStruct((NP, F), jnp.float32),
    )(h, A, S1, S2, MN, MX, d, avg_log, pw, pb.reshape(1, F), lw,
      lb.reshape(1, F))


def _k3_body(h_ref, b_ref, fcw_ref, fcb_ref, fw_ref, fb_ref, out_ref, g_acc):
    i = pl.program_id(0)
    @pl.when(i == 0)
    def _():
        g_acc[...] = jnp.zeros_like(g_acc)
    bt = b_ref[...]
    oh = (bt == lax.broadcasted_iota(jnp.int32, (bt.shape[0], G), 1)
          ).astype(jnp.float32)
    g_acc[...] += lax.dot_general(oh, h_ref[...], (((0,), (0,)), ((), ())),
                                  preferred_element_type=jnp.float32)
    @pl.when(i == pl.num_programs(0) - 1)
    def _():
        g = g_acc[...] @ fcw_ref[...] + fcb_ref[...]
        out_ref[...] = jnp.dot(g, fw_ref[...],
                               preferred_element_type=jnp.float32) + fb_ref[...]


def _k3(h, batch, fcw, fcb, fw, fb):
    blk = 1024
    grid = NP // blk
    return pl.pallas_call(
        _k3_body,
        grid=(grid,),
        in_specs=[
            pl.BlockSpec((blk, F), lambda i: (i, 0)),
            pl.BlockSpec((blk, 1), lambda i: (i, 0)),
            pl.BlockSpec((F, F), lambda i: (0, 0)),
            pl.BlockSpec((1, F), lambda i: (0, 0)),
            pl.BlockSpec((F, OUT), lambda i: (0, 0)),
            pl.BlockSpec((1, OUT), lambda i: (0, 0)),
        ],
        out_specs=pl.BlockSpec((G, OUT), lambda i: (0, 0)),
        out_shape=jax.ShapeDtypeStruct((G, OUT), jnp.float32),
        scratch_shapes=[pltpu.VMEM((G, F), jnp.float32)],
    )(h, batch, fcw, fcb.reshape(1, F), fw, fb.reshape(1, OUT))


def kernel(x, edge_index, batch, pre_W, pre_b, post_W, post_b, lin_W, lin_b,
           fc_W, fc_b, f_W, f_b):
    xp = jnp.pad(x, ((0, NP - N), (0, 0)))
    batchp = jnp.pad(batch, (0, NP - N), constant_values=G).reshape(NP, 1)

    _edges_arr, edges_srt, bmeta, offtab = _sc_partition(
        edge_index[0], edge_index[1])

    h = xp
    d = None
    avg_log = None
    for l in range(L):
        A, B = _k1(h, pre_W[l], pre_b[l])
        S1, S2, MN, MX, DC = _sc_aggregate(B, edges_srt, bmeta, offtab)
        if l == 0:
            d = DC[:, :1]
            avg_log = _k0(d)
        h = _k2(h, A, S1, S2, MN, MX, d, avg_log,
                post_W[l], post_b[l], lin_W[l], lin_b[l], relu=(l < L - 1))
    return _k3(h, batchp, fc_W, fc_b, f_W, f_b)


# speedup vs baseline: 1.3875x; 1.3875x over previous
import functools
import jax
import jax.numpy as jnp
from jax import lax
from jax.experimental import pallas as pl
from jax.experimental.pallas import tpu as pltpu
from jax.experimental.pallas import tpu_sc as plsc

N = 10000
E = 320000
F = 128
OUT = 2048
L = 6
G = 64

NP = 10240
NB = 64
NPB = 160
NWC = 16
NCORE = 2
EPW = E // (NWC * NCORE)
CHA = 2000
NCHA = EPW // CHA
CHB = 128
CHS = 512
WSORT = 2048
OFFR = 168
CAPC = E // 2 + NB * NWC * 8 + CHS
MAGIC = 52429
FBIG = 3.0e38


def _wid():
    return lax.axis_index("s") * NCORE + lax.axis_index("c")


def _sc_partition_body(esrc_hbm, edst_hbm, edges_hbm, srt_hbm, bmeta_hbm,
                       offtab_hbm,
                       src_v, dst_v, stage_v, pub_v, rb_v, meta_v, win_v,
                       shared_v, cnt_s, base_s, fill_s, gst_s, gln_s,
                       cnt2_s, off_s):
    c = lax.axis_index("c")
    s = lax.axis_index("s")
    ebase = (c * NWC + s) * EPW

    def init_cnt(b, _):
        cnt_s[b] = 0
        return 0
    lax.fori_loop(0, NB, init_cnt, 0)

    def count_chunk(ch, _):
        pltpu.sync_copy(edst_hbm.at[pl.ds(ebase + ch * CHA, CHA)], dst_v)
        def grp(g, _):
            d16 = dst_v[pl.ds(g * 16, 16)]
            b16 = lax.shift_right_logical(d16 * MAGIC, 23)
            for j in range(16):
                b = b16[j]
                cnt_s[b] = cnt_s[b] + 1
            return 0
        lax.fori_loop(0, CHA // 16, grp, 0)
        return 0
    lax.fori_loop(0, NCHA, count_chunk, 0)

    def pub(b, _):
        pub_v[b, :] = jnp.full((16,), cnt_s[b], jnp.int32)
        return 0
    lax.fori_loop(0, NB, pub, 0)
    pltpu.sync_copy(pub_v, shared_v.at[s])
    plsc.subcore_barrier()
    pltpu.sync_copy(shared_v, rb_v)

    def bucket_base(b, run):
        def acc_w(w, carry):
            part, tot = carry
            cnt = rb_v[w, b, :][0]
            r8 = (cnt + 7) & (-8)
            part = part + jnp.where(w < s, r8, 0)
            return (part, tot + r8)
        part, tot = lax.fori_loop(0, NWC, acc_w, (0, 0))
        base_s[b] = run + part
        fill_s[b] = 0
        gst_s[b] = run
        gln_s[b] = tot
        meta_v[b, :] = jnp.full((16,), run, jnp.int32)
        meta_v[NB + b, :] = jnp.full((16,), tot, jnp.int32)
        return run + tot
    lax.fori_loop(0, NB, bucket_base, 0)

    @pl.when(s == 0)
    def _():
        mrow = pl.multiple_of(c * 2 * NB, 8)
        pltpu.sync_copy(meta_v, bmeta_hbm.at[pl.ds(mrow, 2 * NB)])

    crow = c * CAPC

    def place_chunk(ch, _):
        pltpu.sync_copy(esrc_hbm.at[pl.ds(ebase + ch * CHA, CHA)], src_v)
        pltpu.sync_copy(edst_hbm.at[pl.ds(ebase + ch * CHA, CHA)], dst_v)
        def grp(g, _):
            s16 = src_v[pl.ds(g * 16, 16)]
            d16 = dst_v[pl.ds(g * 16, 16)]
            b16 = lax.shift_right_logical(d16 * MAGIC, 23)
            p16 = s16 | lax.shift_left(d16, 14)
            for j in range(16):
                b = b16[j]
                pk = p16[j]
                f = fill_s[b]
                stage_v[b * 8 + (f & 7), :] = jnp.full((16,), pk, jnp.int32)
                fill_s[b] = f + 1
                @pl.when((f & 7) == 7)
                def _():
                    row0 = pl.multiple_of(crow + base_s[b] + f - 7, 8)
                    pltpu.sync_copy(stage_v.at[pl.ds(b * 8, 8)],
                                    edges_hbm.at[pl.ds(row0, 8)])
            return 0
        lax.fori_loop(0, CHA // 16, grp, 0)
        return 0
    lax.fori_loop(0, NCHA, place_chunk, 0)

    def tail(b, _):
        f = fill_s[b]
        rem = f & 7
        @pl.when(rem > 0)
        def _():
            def padrow(r, _):
                @pl.when(r >= rem)
                def _():
                    stage_v[b * 8 + r, :] = jnp.full((16,), -1, jnp.int32)
                return 0
            lax.fori_loop(0, 8, padrow, 0)
            row0 = pl.multiple_of(crow + base_s[b] + f - rem, 8)
            pltpu.sync_copy(stage_v.at[pl.ds(b * 8, 8)],
                            edges_hbm.at[pl.ds(row0, 8)])
        return 0
    lax.fori_loop(0, NB, tail, 0)

    plsc.subcore_barrier()
    iota16 = lax.iota(jnp.int32, 16)

    def sort_bucket(bb, _):
        b = s * 4 + bb
        lo = b * NPB
        gstart = gst_s[b]
        glen = gln_s[b]
        nch = (glen + CHS - 1) // CHS

        def zcnt(i, _):
            cnt2_s[i] = 0
            return 0
        lax.fori_loop(0, NPB, zcnt, 0)

        def count_ch(ch, _):
            row0 = pl.multiple_of(crow + gstart + ch * CHS, 8)
            pltpu.sync_copy(edges_hbm.at[pl.ds(row0, CHS)], stage_v)
            nval = glen - ch * CHS
            def cgrp(g, _):
                pk16 = plsc.load_gather(
                    stage_v, [jnp.full((16,), g * 16, jnp.int32) + iota16,
                              jnp.zeros((16,), jnp.int32)])
                ld16 = lax.shift_right_arithmetic(pk16, 14) - lo
                rid = jnp.full((16,), g * 16, jnp.int32) + iota16
                ok16 = ((ld16 >= 0) & (ld16 < NPB)
                        & (rid < nval)).astype(jnp.int32)
                for j in range(16):
                    @pl.when(ok16[j] != 0)
                    def _():
                        ld = ld16[j]
                        cnt2_s[ld] = cnt2_s[ld] + 1
                return 0
            lax.fori_loop(0, CHS // 16, cgrp, 0)
            return 0
        lax.fori_loop(0, nch, count_ch, 0)

        def pref(i, run2):
            off_s[i] = run2
            return run2 + cnt2_s[i]
        vcnt = lax.fori_loop(0, NPB, pref, 0)

        def expoff(i, _):
            win_v[i, :] = jnp.full((16,), off_s[i], jnp.int32)
            return 0
        lax.fori_loop(0, NPB, expoff, 0)
        win_v[NPB, :] = jnp.full((16,), vcnt, jnp.int32)
        orow0 = pl.multiple_of((c * NB + b) * OFFR, 8)
        pltpu.sync_copy(win_v.at[pl.ds(0, OFFR)],
                        offtab_hbm.at[pl.ds(orow0, OFFR)])

        nwin = (glen + WSORT - 1) // WSORT

        def do_win(wnd, _):
            w0 = wnd * WSORT
            def rst(i, _):
                cnt2_s[i] = off_s[i]
                return 0
            lax.fori_loop(0, NPB, rst, 0)
            def prefill(i, _):
                win_v[i, :] = jnp.full((16,), -1, jnp.int32)
                return 0
            lax.fori_loop(0, WSORT, prefill, 0)

            def place_ch(ch, _):
                row0 = pl.multiple_of(crow + gstart + ch * CHS, 8)
                pltpu.sync_copy(edges_hbm.at[pl.ds(row0, CHS)], stage_v)
                nval = glen - ch * CHS
                def pgrp(g, _):
                    pk16 = plsc.load_gather(
                        stage_v, [jnp.full((16,), g * 16, jnp.int32) + iota16,
                                  jnp.zeros((16,), jnp.int32)])
                    ld16 = lax.shift_right_arithmetic(pk16, 14) - lo
                    rid = jnp.full((16,), g * 16, jnp.int32) + iota16
                    ok16 = ((ld16 >= 0) & (ld16 < NPB)
                            & (rid < nval)).astype(jnp.int32)
                    for j in range(16):
                        @pl.when(ok16[j] != 0)
                        def _():
                            ld = ld16[j]
                            pos = cnt2_s[ld]
                            cnt2_s[ld] = pos + 1
                            @pl.when((pos >= w0) & (pos < w0 + WSORT))
                            def _():
                                win_v[pos - w0, :] = jnp.full(
                                    (16,), pk16[j], jnp.int32)
                    return 0
                lax.fori_loop(0, CHS // 16, pgrp, 0)
                return 0
            lax.fori_loop(0, nch, place_ch, 0)

            nrows = jnp.minimum(WSORT, glen - w0)
            orow = crow + gstart + w0
            def w64(k, _):
                r0 = pl.multiple_of(orow + k * 64, 8)
                pltpu.sync_copy(win_v.at[pl.ds(k * 64, 64)],
                                srt_hbm.at[pl.ds(r0, 64)])
                return 0
            n64 = nrows // 64
            lax.fori_loop(0, n64, w64, 0)
            def w8(k, _):
                soff = pl.multiple_of(n64 * 64 + k * 8, 8)
                r0 = pl.multiple_of(orow + n64 * 64 + k * 8, 8)
                pltpu.sync_copy(win_v.at[pl.ds(soff, 8)],
                                srt_hbm.at[pl.ds(r0, 8)])
                return 0
            lax.fori_loop(0, (nrows - n64 * 64) // 8, w8, 0)
            return 0
        lax.fori_loop(0, nwin, do_win, 0)
        return 0
    lax.fori_loop(0, 4, sort_bucket, 0)


_sc_partition = functools.partial(
    pl.kernel,
    out_type=(
        jax.ShapeDtypeStruct((2 * CAPC, 16), jnp.int32),
        jax.ShapeDtypeStruct((2 * CAPC, 16), jnp.int32),
        jax.ShapeDtypeStruct((2 * 2 * NB, 16), jnp.int32),
        jax.ShapeDtypeStruct((2 * NB * OFFR, 16), jnp.int32),
    ),
    mesh=plsc.VectorSubcoreMesh(core_axis_name="c", subcore_axis_name="s"),
    scratch_types=[
        pltpu.VMEM((CHA,), jnp.int32),
        pltpu.VMEM((CHA,), jnp.int32),
        pltpu.VMEM((NB * 8, 16), jnp.int32),
        pltpu.VMEM((NB, 16), jnp.int32),
        pltpu.VMEM((NWC, NB, 16), jnp.int32),
        pltpu.VMEM((2 * NB, 16), jnp.int32),
        pltpu.VMEM((WSORT, 16), jnp.int32),
        pltpu.VMEM_SHARED((NWC, NB, 16), jnp.int32),
        pltpu.SMEM((NB,), jnp.int32),
        pltpu.SMEM((NB,), jnp.int32),
        pltpu.SMEM((NB,), jnp.int32),
        pltpu.SMEM((NB,), jnp.int32),
        pltpu.SMEM((NB,), jnp.int32),
        pltpu.SMEM((NPB,), jnp.int32),
        pltpu.SMEM((NPB,), jnp.int32),
    ],
    compiler_params=pltpu.CompilerParams(use_tc_tiling_on_sc=False, needs_layout_passes=False),
)(_sc_partition_body)


def _sc_aggregate_body(b_hbm, edges_hbm, bmeta_hbm, offtab_hbm,
                       s1_hbm, s2_hbm, mn_hbm, mx_hbm, dc_hbm,
                       acc_v, dc_v, pk_v, gidx_v, rows_v, meta_v, off_v,
                       sem, psem):
    c = lax.axis_index("c")
    s = lax.axis_index("s")
    w = s * NCORE + c
    iota16 = lax.iota(jnp.int32, 16)
    zf = jnp.zeros((16,), jnp.float32)
    bigf = jnp.full((16,), FBIG, jnp.float32)
    pltpu.sync_copy(bmeta_hbm, meta_v)

    id_regs = tuple([zf] * 8 + [zf] * 8 + [bigf] * 8 + [-bigf] * 8)

    def do_bucket(r, _):
        b = w * 2 + r
        lo = pl.multiple_of(b * NPB, 8)

        def initrow(i, _):
            for f in range(8):
                sl = pl.ds(f * 16, 16)
                acc_v[0, i, sl] = zf
                acc_v[1, i, sl] = zf
                acc_v[2, i, sl] = bigf
                acc_v[3, i, sl] = -bigf
            dc_v[i, :] = zf
            return 0
        lax.fori_loop(0, NPB, initrow, 0)

        for c2 in range(NCORE):
            gstart = meta_v[c2 * 2 * NB + b, :][0]
            crow = c2 * CAPC
            orow = pl.multiple_of((c2 * NB + b) * OFFR, 8)
            pltpu.sync_copy(offtab_hbm.at[pl.ds(orow, OFFR)], off_v)
            vcnt = off_v[NPB, :][0]

            def off_at(i):
                return off_v[i, :][0]

            nchunks = (vcnt + CHB - 1) // CHB

            def pk_desc(i):
                row0 = pl.multiple_of(crow + gstart + i * CHB, 8)
                buf = pl.multiple_of((i & 1) * CHB, 8)
                return pltpu.make_async_copy(
                    edges_hbm.at[pl.ds(row0, CHB)],
                    pk_v.at[pl.ds(buf, CHB)], psem)

            def gather_desc(i):
                buf = pl.multiple_of((i & 1) * CHB, 8)
                return pltpu.make_async_copy(
                    b_hbm.at[gidx_v.at[pl.ds(buf, CHB)]],
                    rows_v.at[pl.ds(buf, CHB)], sem)

            def mk_and_gather(i):
                buf = pl.multiple_of((i & 1) * CHB, 8)

                @plsc.parallel_loop(0, CHB // 16, unroll=4)
                def mkidx(g):
                    pkg = plsc.load_gather(
                        pk_v, [buf + jnp.full((16,), g * 16, jnp.int32)
                               + iota16,
                               jnp.zeros((16,), jnp.int32)])
                    srcg = jnp.minimum(pkg & 0x3FFF, NP - 1)
                    gidx_v[pl.ds(buf + g * 16, 16)] = jnp.maximum(srcg, 0)
                gather_desc(i).start()

            @pl.when(nchunks > 0)
            def _():
                pk_desc(0).start()
                pk_desc(0).wait()
                mk_and_gather(0)
                @pl.when(nchunks > 1)
                def _():
                    pk_desc(1).start()

            def do_chunk(ch, node):
                c0 = ch * CHB
                cend = jnp.minimum(c0 + CHB, vcnt)
                rbuf = (ch & 1) * CHB
                gather_desc(ch).wait()

                @pl.when(ch + 1 < nchunks)
                def _():
                    pk_desc(ch + 1).wait()
                    mk_and_gather(ch + 1)

                @pl.when(ch + 2 < nchunks)
                def _():
                    pk_desc(ch + 2).start()

                def wcond(carry):
                    node_, start_, cont_ = carry
                    return cont_ != 0

                def wbody(carry):
                    node_, start_, _ = carry
                    nend = off_at(node_ + 1)
                    seg0 = jnp.maximum(start_, c0)
                    seg1 = jnp.minimum(nend, cend)

                    def acc_edge(e, regs):
                        jr = e - c0 + rbuf
                        nr = []
                        for f in range(8):
                            rv = rows_v[jr, pl.ds(f * 16, 16)]
                            nr.append((regs[f] + rv,
                                       regs[8 + f] + rv * rv,
                                       jnp.minimum(regs[16 + f], rv),
                                       jnp.maximum(regs[24 + f], rv)))
                        return (tuple(t[0] for t in nr)
                                + tuple(t[1] for t in nr)
                                + tuple(t[2] for t in nr)
                                + tuple(t[3] for t in nr))
                    regs = lax.fori_loop(seg0, seg1, acc_edge, id_regs)

                    @pl.when(seg1 > seg0)
                    def _():
                        for f in range(8):
                            sl = pl.ds(f * 16, 16)
                            plsc.addupdate(acc_v.at[0, node_, sl], regs[f])
                            plsc.addupdate(acc_v.at[1, node_, sl], regs[8 + f])
                            mnv = acc_v[2, node_, sl]
                            acc_v[2, node_, sl] = jnp.minimum(mnv, regs[16 + f])
                            mxv = acc_v[3, node_, sl]
                            acc_v[3, node_, sl] = jnp.maximum(mxv, regs[24 + f])
                        plsc.addupdate(
                            dc_v.at[node_],
                            zf + (seg1 - seg0).astype(jnp.float32))

                    adv = (nend <= cend).astype(jnp.int32)
                    node2 = jnp.where(adv != 0, node_ + 1, node_)
                    start2 = jnp.where(adv != 0, nend, start_)
                    cont2 = jnp.where(
                        (adv != 0) & (node2 < NPB) & (start2 < cend), 1, 0)
                    return (node2, start2, cont2)

                start0 = off_at(node)
                cont0 = jnp.where((node < NPB) & (start0 < cend), 1, 0)
                node, _, _ = lax.while_loop(wcond, wbody,
                                            (node, start0, cont0))
                return node

            lax.fori_loop(0, nchunks, do_chunk, jnp.int32(0))

        pltpu.sync_copy(acc_v.at[0], s1_hbm.at[pl.ds(lo, NPB)])
        pltpu.sync_copy(acc_v.at[1], s2_hbm.at[pl.ds(lo, NPB)])
        pltpu.sync_copy(acc_v.at[2], mn_hbm.at[pl.ds(lo, NPB)])
        pltpu.sync_copy(acc_v.at[3], mx_hbm.at[pl.ds(lo, NPB)])
        pltpu.sync_copy(dc_v, dc_hbm.at[pl.ds(lo, NPB)])
        return 0
    lax.fori_loop(0, 2, do_bucket, 0)


_sc_aggregate = functools.partial(
    pl.kernel,
    out_type=(
        jax.ShapeDtypeStruct((NP, F), jnp.float32),
        jax.ShapeDtypeStruct((NP, F), jnp.float32),
        jax.ShapeDtypeStruct((NP, F), jnp.float32),
        jax.ShapeDtypeStruct((NP, F), jnp.float32),
        jax.ShapeDtypeStruct((NP, 16), jnp.float32),
    ),
    mesh=plsc.VectorSubcoreMesh(core_axis_name="c", subcore_axis_name="s"),
    scratch_types=[
        pltpu.VMEM((4, NPB, F), jnp.float32),
        pltpu.VMEM((NPB, 16), jnp.float32),
        pltpu.VMEM((2 * CHB, 16), jnp.int32),
        pltpu.VMEM((2 * CHB,), jnp.int32),
        pltpu.VMEM((2 * CHB, F), jnp.float32),
        pltpu.VMEM((2 * 2 * NB, 16), jnp.int32),
        pltpu.VMEM((OFFR, 16), jnp.int32),
        pltpu.SemaphoreType.DMA,
        pltpu.SemaphoreType.DMA,
    ],
    compiler_params=pltpu.CompilerParams(use_tc_tiling_on_sc=False, needs_layout_passes=False),
)(_sc_aggregate_body)


def _k1_body(h_ref, w_ref, b_ref, a_out, b_out):
    h = h_ref[...]
    a_out[...] = jnp.dot(h, w_ref[0:F, :], preferred_element_type=jnp.float32)
    b_out[...] = jnp.dot(h, w_ref[F:2 * F, :],
                         preferred_element_type=jnp.float32) + b_ref[...]


def _k1(h, w, b):
    blk = 1024
    grid = NP // blk
    return pl.pallas_call(
        _k1_body,
        grid=(grid,),
        in_specs=[
            pl.BlockSpec((blk, F), lambda i: (i, 0)),
            pl.BlockSpec((2 * F, F), lambda i: (0, 0)),
            pl.BlockSpec((1, F), lambda i: (0, 0)),
        ],
        out_specs=[
            pl.BlockSpec((blk, F), lambda i: (i, 0)),
            pl.BlockSpec((blk, F), lambda i: (i, 0)),
        ],
        out_shape=[
            jax.ShapeDtypeStruct((NP, F), jnp.float32),
            jax.ShapeDtypeStruct((NP, F), jnp.float32),
        ],
    )(h, w, b.reshape(1, F))


def _k0_body(d_ref, out_ref):
    d = d_ref[...]
    out_ref[...] = (jnp.sum(jnp.log(d + 1.0)) / N).reshape(1, 1)


def _k0(d):
    return pl.pallas_call(
        _k0_body,
        out_shape=jax.ShapeDtypeStruct((1, 1), jnp.float32),
    )(d)


def _k2_body(h_ref, a_ref, s1_ref, s2_ref, mn_ref, mx_ref, d_ref, al_ref,
             pw_ref, pb_ref, lw_ref, lb_ref, out_ref, *, relu):
    h = h_ref[...]
    A = a_ref[...]
    S1 = s1_ref[...]
    S2 = s2_ref[...]
    d = d_ref[...]
    avg_log = al_ref[0, 0]
    deg_c = jnp.maximum(d, 1.0)
    has = (d > 0).astype(jnp.float32)
    mean = (d * A + S1) / deg_c
    mean_sq = (d * A * A + 2.0 * A * S1 + S2) / deg_c
    std = jnp.sqrt(jax.nn.relu(mean_sq - mean * mean) + 1e-5)
    mn = has * (A + jnp.where(d > 0, mn_ref[...], 0.0))
    mx = has * (A + jnp.where(d > 0, mx_ref[...], 0.0))
    lg = jnp.log(deg_c + 1.0)
    amp = lg / avg_log
    att = avg_log / lg
    parts = [h, mean, mn, mx, std,
             amp * mean, amp * mn, amp * mx, amp * std,
             att * mean, att * mn, att * mx, att * std]
    acc = pb_ref[...].astype(jnp.float32)
    o = jnp.zeros_like(h) + acc
    for i, p in enumerate(parts):
        o = o + jnp.dot(p, pw_ref[i * F:(i + 1) * F, :],
                        preferred_element_type=jnp.float32)
    o = jnp.dot(o, lw_ref[...], preferred_element_type=jnp.float32) + lb_ref[...]
    if relu:
        o = jax.nn.relu(o)
    out_ref[...] = o


def _k2(h, A, S1, S2, MN, MX, d, avg_log, pw, pb, lw, lb, relu):
    blk = 512
    grid = NP // blk
    body = functools.partial(_k2_body, relu=relu)
    return pl.pallas_call(
        body,
        grid=(grid,),
        in_specs=[
            pl.BlockSpec((blk, F), lambda i: (i, 0)),
            pl.BlockSpec((blk, F), lambda i: (i, 0)),
            pl.BlockSpec((blk, F), lambda i: (i, 0)),
            pl.BlockSpec((blk, F), lambda i: (i, 0)),
            pl.BlockSpec((blk, F), lambda i: (i, 0)),
            pl.BlockSpec((blk, F), lambda i: (i, 0)),
            pl.BlockSpec((blk, 1), lambda i: (i, 0)),
            pl.BlockSpec((1, 1), lambda i: (0, 0)),
            pl.BlockSpec((13 * F, F), lambda i: (0, 0)),
            pl.BlockSpec((1, F), lambda i: (0, 0)),
            pl.BlockSpec((F, F), lambda i: (0, 0)),
            pl.BlockSpec((1, F), lambda i: (0, 0)),
        ],
        out_specs=pl.BlockSpec((blk, F), lambda i: (i, 0)),
        out_shape=jax.ShapeDtypeStruct((NP, F), jnp.float32),
    )(h, A, S1, S2, MN, MX, d, avg_log, pw, pb.reshape(1, F), lw,
      lb.reshape(1, F))


def _k3_body(h_ref, b_ref, fcw_ref, fcb_ref, fw_ref, fb_ref, out_ref, g_acc):
    i = pl.program_id(0)
    @pl.when(i == 0)
    def _():
        g_acc[...] = jnp.zeros_like(g_acc)
    bt = b_ref[...]
    oh = (bt == lax.broadcasted_iota(jnp.int32, (bt.shape[0], G), 1)
          ).astype(jnp.float32)
    g_acc[...] += lax.dot_general(oh, h_ref[...], (((0,), (0,)), ((), ())),
                                  preferred_element_type=jnp.float32)
    @pl.when(i == pl.num_programs(0) - 1)
    def _():
        g = g_acc[...] @ fcw_ref[...] + fcb_ref[...]
        out_ref[...] = jnp.dot(g, fw_ref[...],
                               preferred_element_type=jnp.float32) + fb_ref[...]


def _k3(h, batch, fcw, fcb, fw, fb):
    blk = 1024
    grid = NP // blk
    return pl.pallas_call(
        _k3_body,
        grid=(grid,),
        in_specs=[
            pl.BlockSpec((blk, F), lambda i: (i, 0)),
            pl.BlockSpec((blk, 1), lambda i: (i, 0)),
            pl.BlockSpec((F, F), lambda i: (0, 0)),
            pl.BlockSpec((1, F), lambda i: (0, 0)),
            pl.BlockSpec((F, OUT), lambda i: (0, 0)),
            pl.BlockSpec((1, OUT), lambda i: (0, 0)),
        ],
        out_specs=pl.BlockSpec((G, OUT), lambda i: (0, 0)),
        out_shape=jax.ShapeDtypeStruct((G, OUT), jnp.float32),
        scratch_shapes=[pltpu.VMEM((G, F), jnp.float32)],
    )(h, batch, fcw, fcb.reshape(1, F), fw, fb.reshape(1, OUT))


def kernel(x, edge_index, batch, pre_W, pre_b, post_W, post_b, lin_W, lin_b,
           fc_W, fc_b, f_W, f_b):
    xp = jnp.pad(x, ((0, NP - N), (0, 0)))
    batchp = jnp.pad(batch, (0, NP - N), constant_values=G).reshape(NP, 1)

    _edges_arr, edges_srt, bmeta, offtab = _sc_partition(
        edge_index[0], edge_index[1])

    h = xp
    d = None
    avg_log = None
    for l in range(L):
        A, B = _k1(h, pre_W[l], pre_b[l])
        S1, S2, MN, MX, DC = _sc_aggregate(B, edges_srt, bmeta, offtab)
        if l == 0:
            d = DC[:, :1]
            avg_log = _k0(d)
        h = _k2(h, A, S1, S2, MN, MX, d, avg_log,
                post_W[l], post_b[l], lin_W[l], lin_b[l], relu=(l < L - 1))
    return _k3(h, batchp, fc_W, fc_b, f_W, f_b)


# speedup vs baseline: 1.4161x; 1.0207x over previous
import functools
import jax
import jax.numpy as jnp
from jax import lax
from jax.experimental import pallas as pl
from jax.experimental.pallas import tpu as pltpu
from jax.experimental.pallas import tpu_sc as plsc

N = 10000
E = 320000
F = 128
OUT = 2048
L = 6
G = 64

NP = 10240
NB = 64
NPB = 160
NWC = 16
NCORE = 2
EPW = E // (NWC * NCORE)
CHA = 2000
NCHA = EPW // CHA
CHB = 128
CHS = 512
WSORT = 2048
OFFR = 168
CAPC = E // 2 + NB * NWC * 8 + CHS
MAGIC = 52429
FBIG = 3.0e38


def _wid():
    return lax.axis_index("s") * NCORE + lax.axis_index("c")


def _sc_partition_body(esrc_hbm, edst_hbm, edges_hbm, srt_hbm, bmeta_hbm,
                       offtab_hbm,
                       src_v, dst_v, stage_v, pub_v, rb_v, meta_v, win_v,
                       hist_v, hist2_v,
                       shared_v, base_s, fill_s, gst_s, gln_s,
                       cnt2_s, off_s):
    c = lax.axis_index("c")
    s = lax.axis_index("s")
    ebase = (c * NWC + s) * EPW

    iota16a = lax.iota(jnp.int32, 16)
    onesi = jnp.full((16,), 1, jnp.int32)

    def zhist(i, _):
        hist_v[pl.ds(i * 16, 16)] = jnp.zeros((16,), jnp.int32)
        return 0
    lax.fori_loop(0, NB, zhist, 0)

    def count_chunk(ch, _):
        pltpu.sync_copy(edst_hbm.at[pl.ds(ebase + ch * CHA, CHA)], dst_v)
        def grp(g, _):
            d16 = dst_v[pl.ds(g * 16, 16)]
            b16 = lax.shift_right_logical(d16 * MAGIC, 23)
            plsc.addupdate_scatter(hist_v, [b16 * 16 + iota16a], onesi)
            return 0
        lax.fori_loop(0, CHA // 16, grp, 0)
        return 0
    lax.fori_loop(0, NCHA, count_chunk, 0)

    def pub(b, _):
        cnt = jnp.sum(hist_v[pl.ds(b * 16, 16)])
        pub_v[b, :] = jnp.zeros((16,), jnp.int32) + cnt
        return 0
    lax.fori_loop(0, NB, pub, 0)
    pltpu.sync_copy(pub_v, shared_v.at[s])
    plsc.subcore_barrier()
    pltpu.sync_copy(shared_v, rb_v)

    def bucket_base(b, run):
        def acc_w(w, carry):
            part, tot = carry
            cnt = rb_v[w, b, :][0]
            r8 = (cnt + 7) & (-8)
            part = part + jnp.where(w < s, r8, 0)
            return (part, tot + r8)
        part, tot = lax.fori_loop(0, NWC, acc_w, (0, 0))
        base_s[b] = run + part
        fill_s[b] = 0
        gst_s[b] = run
        gln_s[b] = tot
        meta_v[b, :] = jnp.full((16,), run, jnp.int32)
        meta_v[NB + b, :] = jnp.full((16,), tot, jnp.int32)
        return run + tot
    lax.fori_loop(0, NB, bucket_base, 0)

    @pl.when(s == 0)
    def _():
        mrow = pl.multiple_of(c * 2 * NB, 8)
        pltpu.sync_copy(meta_v, bmeta_hbm.at[pl.ds(mrow, 2 * NB)])

    crow = c * CAPC

    def place_chunk(ch, _):
        pltpu.sync_copy(esrc_hbm.at[pl.ds(ebase + ch * CHA, CHA)], src_v)
        pltpu.sync_copy(edst_hbm.at[pl.ds(ebase + ch * CHA, CHA)], dst_v)
        def grp(g, _):
            s16 = src_v[pl.ds(g * 16, 16)]
            d16 = dst_v[pl.ds(g * 16, 16)]
            b16 = lax.shift_right_logical(d16 * MAGIC, 23)
            p16 = s16 | lax.shift_left(d16, 14)
            for j in range(16):
                b = b16[j]
                pk = p16[j]
                f = fill_s[b]
                stage_v[b * 8 + (f & 7), :] = jnp.full((16,), pk, jnp.int32)
                fill_s[b] = f + 1
                @pl.when((f & 7) == 7)
                def _():
                    row0 = pl.multiple_of(crow + base_s[b] + f - 7, 8)
                    pltpu.sync_copy(stage_v.at[pl.ds(b * 8, 8)],
                                    edges_hbm.at[pl.ds(row0, 8)])
            return 0
        lax.fori_loop(0, CHA // 16, grp, 0)
        return 0
    lax.fori_loop(0, NCHA, place_chunk, 0)

    def tail(b, _):
        f = fill_s[b]
        rem = f & 7
        @pl.when(rem > 0)
        def _():
            def padrow(r, _):
                @pl.when(r >= rem)
                def _():
                    stage_v[b * 8 + r, :] = jnp.full((16,), -1, jnp.int32)
                return 0
            lax.fori_loop(0, 8, padrow, 0)
            row0 = pl.multiple_of(crow + base_s[b] + f - rem, 8)
            pltpu.sync_copy(stage_v.at[pl.ds(b * 8, 8)],
                            edges_hbm.at[pl.ds(row0, 8)])
        return 0
    lax.fori_loop(0, NB, tail, 0)

    plsc.subcore_barrier()
    iota16 = lax.iota(jnp.int32, 16)

    def sort_bucket(bb, _):
        b = s * 4 + bb
        lo = b * NPB
        gstart = gst_s[b]
        glen = gln_s[b]
        nch = (glen + CHS - 1) // CHS

        def zh2(i, _):
            hist2_v[pl.ds(i * 16, 16)] = jnp.zeros((16,), jnp.int32)
            return 0
        lax.fori_loop(0, NPB + 1, zh2, 0)

        def count_ch(ch, _):
            row0 = pl.multiple_of(crow + gstart + ch * CHS, 8)
            pltpu.sync_copy(edges_hbm.at[pl.ds(row0, CHS)], stage_v)
            nval = glen - ch * CHS
            def cgrp(g, _):
                pk16 = plsc.load_gather(
                    stage_v, [jnp.full((16,), g * 16, jnp.int32) + iota16,
                              jnp.zeros((16,), jnp.int32)])
                ld16 = lax.shift_right_arithmetic(pk16, 14) - lo
                rid = jnp.full((16,), g * 16, jnp.int32) + iota16
                ok16 = (ld16 >= 0) & (ld16 < NPB) & (rid < nval)
                idx = jnp.where(ok16, ld16 * 16, NPB * 16) + iota16
                plsc.addupdate_scatter(hist2_v, [idx], onesi)
                return 0
            lax.fori_loop(0, CHS // 16, cgrp, 0)
            return 0
        lax.fori_loop(0, nch, count_ch, 0)

        def pref(i, run2):
            off_s[i] = run2
            return run2 + jnp.sum(hist2_v[pl.ds(i * 16, 16)])
        vcnt = lax.fori_loop(0, NPB, pref, 0)

        def expoff(i, _):
            win_v[i, :] = jnp.full((16,), off_s[i], jnp.int32)
            return 0
        lax.fori_loop(0, NPB, expoff, 0)
        win_v[NPB, :] = jnp.full((16,), vcnt, jnp.int32)
        orow0 = pl.multiple_of((c * NB + b) * OFFR, 8)
        pltpu.sync_copy(win_v.at[pl.ds(0, OFFR)],
                        offtab_hbm.at[pl.ds(orow0, OFFR)])

        nwin = (glen + WSORT - 1) // WSORT

        def do_win(wnd, _):
            w0 = wnd * WSORT
            def rst(i, _):
                cnt2_s[i] = off_s[i]
                return 0
            lax.fori_loop(0, NPB, rst, 0)
            def prefill(i, _):
                win_v[i, :] = jnp.full((16,), -1, jnp.int32)
                return 0
            lax.fori_loop(0, WSORT, prefill, 0)

            def place_ch(ch, _):
                row0 = pl.multiple_of(crow + gstart + ch * CHS, 8)
                pltpu.sync_copy(edges_hbm.at[pl.ds(row0, CHS)], stage_v)
                nval = glen - ch * CHS
                def pgrp(g, _):
                    pk16 = plsc.load_gather(
                        stage_v, [jnp.full((16,), g * 16, jnp.int32) + iota16,
                                  jnp.zeros((16,), jnp.int32)])
                    ld16 = lax.shift_right_arithmetic(pk16, 14) - lo
                    rid = jnp.full((16,), g * 16, jnp.int32) + iota16
                    ok16 = ((ld16 >= 0) & (ld16 < NPB)
                            & (rid < nval)).astype(jnp.int32)
                    for j in range(16):
                        @pl.when(ok16[j] != 0)
                        def _():
                            ld = ld16[j]
                            pos = cnt2_s[ld]
                            cnt2_s[ld] = pos + 1
                            @pl.when((pos >= w0) & (pos < w0 + WSORT))
                            def _():
                                win_v[pos - w0, :] = jnp.full(
                                    (16,), pk16[j], jnp.int32)
                    return 0
                lax.fori_loop(0, CHS // 16, pgrp, 0)
                return 0
            lax.fori_loop(0, nch, place_ch, 0)

            nrows = jnp.minimum(WSORT, glen - w0)
            orow = crow + gstart + w0
            def w64(k, _):
                r0 = pl.multiple_of(orow + k * 64, 8)
                pltpu.sync_copy(win_v.at[pl.ds(k * 64, 64)],
                                srt_hbm.at[pl.ds(r0, 64)])
                return 0
            n64 = nrows // 64
            lax.fori_loop(0, n64, w64, 0)
            def w8(k, _):
                soff = pl.multiple_of(n64 * 64 + k * 8, 8)
                r0 = pl.multiple_of(orow + n64 * 64 + k * 8, 8)
                pltpu.sync_copy(win_v.at[pl.ds(soff, 8)],
                                srt_hbm.at[pl.ds(r0, 8)])
                return 0
            lax.fori_loop(0, (nrows - n64 * 64) // 8, w8, 0)
            return 0
        lax.fori_loop(0, nwin, do_win, 0)
        return 0
    lax.fori_loop(0, 4, sort_bucket, 0)


_sc_partition = functools.partial(
    pl.kernel,
    out_type=(
        jax.ShapeDtypeStruct((2 * CAPC, 16), jnp.int32),
        jax.ShapeDtypeStruct((2 * CAPC, 16), jnp.int32),
        jax.ShapeDtypeStruct((2 * 2 * NB, 16), jnp.int32),
        jax.ShapeDtypeStruct((2 * NB * OFFR, 16), jnp.int32),
    ),
    mesh=plsc.VectorSubcoreMesh(core_axis_name="c", subcore_axis_name="s"),
    scratch_types=[
        pltpu.VMEM((CHA,), jnp.int32),
        pltpu.VMEM((CHA,), jnp.int32),
        pltpu.VMEM((NB * 8, 16), jnp.int32),
        pltpu.VMEM((NB, 16), jnp.int32),
        pltpu.VMEM((NWC, NB, 16), jnp.int32),
        pltpu.VMEM((2 * NB, 16), jnp.int32),
        pltpu.VMEM((WSORT, 16), jnp.int32),
        pltpu.VMEM((NB * 16,), jnp.int32),
        pltpu.VMEM(((NPB + 1) * 16,), jnp.int32),
        pltpu.VMEM_SHARED((NWC, NB, 16), jnp.int32),
        pltpu.SMEM((NB,), jnp.int32),
        pltpu.SMEM((NB,), jnp.int32),
        pltpu.SMEM((NB,), jnp.int32),
        pltpu.SMEM((NB,), jnp.int32),
        pltpu.SMEM((NPB,), jnp.int32),
        pltpu.SMEM((NPB,), jnp.int32),
    ],
    compiler_params=pltpu.CompilerParams(use_tc_tiling_on_sc=False, needs_layout_passes=False),
)(_sc_partition_body)


def _sc_aggregate_body(b_hbm, edges_hbm, bmeta_hbm, offtab_hbm,
                       s1_hbm, s2_hbm, mn_hbm, mx_hbm, dc_hbm,
                       acc_v, dc_v, pk_v, gidx_v, rows_v, meta_v, off_v,
                       sem, psem):
    c = lax.axis_index("c")
    s = lax.axis_index("s")
    w = s * NCORE + c
    iota16 = lax.iota(jnp.int32, 16)
    zf = jnp.zeros((16,), jnp.float32)
    bigf = jnp.full((16,), FBIG, jnp.float32)
    pltpu.sync_copy(bmeta_hbm, meta_v)

    id_regs = tuple([zf] * 8 + [zf] * 8 + [bigf] * 8 + [-bigf] * 8)

    def do_bucket(r, _):
        b = w * 2 + r
        lo = pl.multiple_of(b * NPB, 8)

        def initrow(i, _):
            for f in range(8):
                sl = pl.ds(f * 16, 16)
                acc_v[0, i, sl] = zf
                acc_v[1, i, sl] = zf
                acc_v[2, i, sl] = bigf
                acc_v[3, i, sl] = -bigf
            dc_v[i, :] = zf
            return 0
        lax.fori_loop(0, NPB, initrow, 0)

        for c2 in range(NCORE):
            gstart = meta_v[c2 * 2 * NB + b, :][0]
            crow = c2 * CAPC
            orow = pl.multiple_of((c2 * NB + b) * OFFR, 8)
            pltpu.sync_copy(offtab_hbm.at[pl.ds(orow, OFFR)], off_v)
            vcnt = off_v[NPB, :][0]

            def off_at(i):
                return off_v[i, :][0]

            nchunks = (vcnt + CHB - 1) // CHB

            def pk_desc(i):
                row0 = pl.multiple_of(crow + gstart + i * CHB, 8)
                buf = pl.multiple_of((i & 1) * CHB, 8)
                return pltpu.make_async_copy(
                    edges_hbm.at[pl.ds(row0, CHB)],
                    pk_v.at[pl.ds(buf, CHB)], psem)

            def gather_desc(i):
                buf = pl.multiple_of((i & 1) * CHB, 8)
                return pltpu.make_async_copy(
                    b_hbm.at[gidx_v.at[pl.ds(buf, CHB)]],
                    rows_v.at[pl.ds(buf, CHB)], sem)

            def mk_and_gather(i):
                buf = pl.multiple_of((i & 1) * CHB, 8)

                @plsc.parallel_loop(0, CHB // 16, unroll=4)
                def mkidx(g):
                    pkg = plsc.load_gather(
                        pk_v, [buf + jnp.full((16,), g * 16, jnp.int32)
                               + iota16,
                               jnp.zeros((16,), jnp.int32)])
                    srcg = jnp.minimum(pkg & 0x3FFF, NP - 1)
                    gidx_v[pl.ds(buf + g * 16, 16)] = jnp.maximum(srcg, 0)
                gather_desc(i).start()

            @pl.when(nchunks > 0)
            def _():
                pk_desc(0).start()
                pk_desc(0).wait()
                mk_and_gather(0)
                @pl.when(nchunks > 1)
                def _():
                    pk_desc(1).start()

            def do_chunk(ch, node):
                c0 = ch * CHB
                cend = jnp.minimum(c0 + CHB, vcnt)
                rbuf = (ch & 1) * CHB
                gather_desc(ch).wait()

                @pl.when(ch + 1 < nchunks)
                def _():
                    pk_desc(ch + 1).wait()
                    mk_and_gather(ch + 1)

                @pl.when(ch + 2 < nchunks)
                def _():
                    pk_desc(ch + 2).start()

                def wcond(carry):
                    node_, start_, cont_ = carry
                    return cont_ != 0

                def wbody(carry):
                    node_, start_, _ = carry
                    nend = off_at(node_ + 1)
                    seg0 = jnp.maximum(start_, c0)
                    seg1 = jnp.minimum(nend, cend)

                    def acc_edge(e, regs):
                        jr = e - c0 + rbuf
                        nr = []
                        for f in range(8):
                            rv = rows_v[jr, pl.ds(f * 16, 16)]
                            nr.append((regs[f] + rv,
                                       regs[8 + f] + rv * rv,
                                       jnp.minimum(regs[16 + f], rv),
                                       jnp.maximum(regs[24 + f], rv)))
                        return (tuple(t[0] for t in nr)
                                + tuple(t[1] for t in nr)
                                + tuple(t[2] for t in nr)
                                + tuple(t[3] for t in nr))
                    regs = lax.fori_loop(seg0, seg1, acc_edge, id_regs)

                    @pl.when(seg1 > seg0)
                    def _():
                        for f in range(8):
                            sl = pl.ds(f * 16, 16)
                            plsc.addupdate(acc_v.at[0, node_, sl], regs[f])
                            plsc.addupdate(acc_v.at[1, node_, sl], regs[8 + f])
                            mnv = acc_v[2, node_, sl]
                            acc_v[2, node_, sl] = jnp.minimum(mnv, regs[16 + f])
                            mxv = acc_v[3, node_, sl]
                            acc_v[3, node_, sl] = jnp.maximum(mxv, regs[24 + f])
                        plsc.addupdate(
                            dc_v.at[node_],
                            zf + (seg1 - seg0).astype(jnp.float32))

                    adv = (nend <= cend).astype(jnp.int32)
                    node2 = jnp.where(adv != 0, node_ + 1, node_)
                    start2 = jnp.where(adv != 0, nend, start_)
                    cont2 = jnp.where(
                        (adv != 0) & (node2 < NPB) & (start2 < cend), 1, 0)
                    return (node2, start2, cont2)

                start0 = off_at(node)
                cont0 = jnp.where((node < NPB) & (start0 < cend), 1, 0)
                node, _, _ = lax.while_loop(wcond, wbody,
                                            (node, start0, cont0))
                return node

            lax.fori_loop(0, nchunks, do_chunk, jnp.int32(0))

        pltpu.sync_copy(acc_v.at[0], s1_hbm.at[pl.ds(lo, NPB)])
        pltpu.sync_copy(acc_v.at[1], s2_hbm.at[pl.ds(lo, NPB)])
        pltpu.sync_copy(acc_v.at[2], mn_hbm.at[pl.ds(lo, NPB)])
        pltpu.sync_copy(acc_v.at[3], mx_hbm.at[pl.ds(lo, NPB)])
        pltpu.sync_copy(dc_v, dc_hbm.at[pl.ds(lo, NPB)])
        return 0
    lax.fori_loop(0, 2, do_bucket, 0)


_sc_aggregate = functools.partial(
    pl.kernel,
    out_type=(
        jax.ShapeDtypeStruct((NP, F), jnp.float32),
        jax.ShapeDtypeStruct((NP, F), jnp.float32),
        jax.ShapeDtypeStruct((NP, F), jnp.float32),
        jax.ShapeDtypeStruct((NP, F), jnp.float32),
        jax.ShapeDtypeStruct((NP, 16), jnp.float32),
    ),
    mesh=plsc.VectorSubcoreMesh(core_axis_name="c", subcore_axis_name="s"),
    scratch_types=[
        pltpu.VMEM((4, NPB, F), jnp.float32),
        pltpu.VMEM((NPB, 16), jnp.float32),
        pltpu.VMEM((2 * CHB, 16), jnp.int32),
        pltpu.VMEM((2 * CHB,), jnp.int32),
        pltpu.VMEM((2 * CHB, F), jnp.float32),
        pltpu.VMEM((2 * 2 * NB, 16), jnp.int32),
        pltpu.VMEM((OFFR, 16), jnp.int32),
        pltpu.SemaphoreType.DMA,
        pltpu.SemaphoreType.DMA,
    ],
    compiler_params=pltpu.CompilerParams(use_tc_tiling_on_sc=False, needs_layout_passes=False),
)(_sc_aggregate_body)


def _k1_body(h_ref, w_ref, b_ref, a_out, b_out):
    h = h_ref[...]
    a_out[...] = jnp.dot(h, w_ref[0:F, :], preferred_element_type=jnp.float32)
    b_out[...] = jnp.dot(h, w_ref[F:2 * F, :],
                         preferred_element_type=jnp.float32) + b_ref[...]


def _k1(h, w, b):
    blk = 1024
    grid = NP // blk
    return pl.pallas_call(
        _k1_body,
        grid=(grid,),
        in_specs=[
            pl.BlockSpec((blk, F), lambda i: (i, 0)),
            pl.BlockSpec((2 * F, F), lambda i: (0, 0)),
            pl.BlockSpec((1, F), lambda i: (0, 0)),
        ],
        out_specs=[
            pl.BlockSpec((blk, F), lambda i: (i, 0)),
            pl.BlockSpec((blk, F), lambda i: (i, 0)),
        ],
        out_shape=[
            jax.ShapeDtypeStruct((NP, F), jnp.float32),
            jax.ShapeDtypeStruct((NP, F), jnp.float32),
        ],
    )(h, w, b.reshape(1, F))


def _k0_body(d_ref, out_ref):
    d = d_ref[...]
    out_ref[...] = (jnp.sum(jnp.log(d + 1.0)) / N).reshape(1, 1)


def _k0(d):
    return pl.pallas_call(
        _k0_body,
        out_shape=jax.ShapeDtypeStruct((1, 1), jnp.float32),
    )(d)


def _k2_body(h_ref, a_ref, s1_ref, s2_ref, mn_ref, mx_ref, d_ref, al_ref,
             pw_ref, pb_ref, lw_ref, lb_ref, out_ref, *, relu):
    _k2_common(h_ref, a_ref, s1_ref, s2_ref, mn_ref, mx_ref, d_ref, al_ref,
               pw_ref, pb_ref, lw_ref, lb_ref, out_ref, None, None, None,
               None, relu=relu)


def _k2f_body(h_ref, a_ref, s1_ref, s2_ref, mn_ref, mx_ref, d_ref, al_ref,
              pw_ref, pb_ref, lw_ref, lb_ref, nw_ref, nb_ref,
              out_ref, na_out, nb_out, *, relu):
    _k2_common(h_ref, a_ref, s1_ref, s2_ref, mn_ref, mx_ref, d_ref, al_ref,
               pw_ref, pb_ref, lw_ref, lb_ref, out_ref, nw_ref, nb_ref,
               na_out, nb_out, relu=relu)


def _k2_common(h_ref, a_ref, s1_ref, s2_ref, mn_ref, mx_ref, d_ref, al_ref,
               pw_ref, pb_ref, lw_ref, lb_ref, out_ref, nw_ref, nb_ref,
               na_out, nb_out, *, relu):
    h = h_ref[...]
    A = a_ref[...]
    S1 = s1_ref[...]
    S2 = s2_ref[...]
    d = d_ref[...]
    avg_log = al_ref[0, 0]
    deg_c = jnp.maximum(d, 1.0)
    has = (d > 0).astype(jnp.float32)
    mean = (d * A + S1) / deg_c
    mean_sq = (d * A * A + 2.0 * A * S1 + S2) / deg_c
    std = jnp.sqrt(jax.nn.relu(mean_sq - mean * mean) + 1e-5)
    mn = has * (A + jnp.where(d > 0, mn_ref[...], 0.0))
    mx = has * (A + jnp.where(d > 0, mx_ref[...], 0.0))
    lg = jnp.log(deg_c + 1.0)
    amp = lg / avg_log
    att = avg_log / lg
    parts = [h, mean, mn, mx, std,
             amp * mean, amp * mn, amp * mx, amp * std,
             att * mean, att * mn, att * mx, att * std]
    acc = pb_ref[...].astype(jnp.float32)
    o = jnp.zeros_like(h) + acc
    for i, p in enumerate(parts):
        o = o + jnp.dot(p, pw_ref[i * F:(i + 1) * F, :],
                        preferred_element_type=jnp.float32)
    o = jnp.dot(o, lw_ref[...], preferred_element_type=jnp.float32) + lb_ref[...]
    if relu:
        o = jax.nn.relu(o)
    out_ref[...] = o
    if nw_ref is not None:
        na_out[...] = jnp.dot(o, nw_ref[0:F, :],
                              preferred_element_type=jnp.float32)
        nb_out[...] = jnp.dot(o, nw_ref[F:2 * F, :],
                              preferred_element_type=jnp.float32) + nb_ref[...]


def _k2f(h, A, S1, S2, MN, MX, d, avg_log, pw, pb, lw, lb, nw, nb):
    blk = 512
    grid = NP // blk
    body = functools.partial(_k2f_body, relu=True)
    rowspec = pl.BlockSpec((blk, F), lambda i: (i, 0))
    return pl.pallas_call(
        body,
        grid=(grid,),
        in_specs=[
            rowspec, rowspec, rowspec, rowspec, rowspec, rowspec,
            pl.BlockSpec((blk, 1), lambda i: (i, 0)),
            pl.BlockSpec((1, 1), lambda i: (0, 0)),
            pl.BlockSpec((13 * F, F), lambda i: (0, 0)),
            pl.BlockSpec((1, F), lambda i: (0, 0)),
            pl.BlockSpec((F, F), lambda i: (0, 0)),
            pl.BlockSpec((1, F), lambda i: (0, 0)),
            pl.BlockSpec((2 * F, F), lambda i: (0, 0)),
            pl.BlockSpec((1, F), lambda i: (0, 0)),
        ],
        out_specs=[rowspec, rowspec, rowspec],
        out_shape=[
            jax.ShapeDtypeStruct((NP, F), jnp.float32),
            jax.ShapeDtypeStruct((NP, F), jnp.float32),
            jax.ShapeDtypeStruct((NP, F), jnp.float32),
        ],
    )(h, A, S1, S2, MN, MX, d, avg_log, pw, pb.reshape(1, F), lw,
      lb.reshape(1, F), nw, nb.reshape(1, F))


def _k2(h, A, S1, S2, MN, MX, d, avg_log, pw, pb, lw, lb, relu):
    blk = 512
    grid = NP // blk
    body = functools.partial(_k2_body, relu=relu)
    return pl.pallas_call(
        body,
        grid=(grid,),
        in_specs=[
            pl.BlockSpec((blk, F), lambda i: (i, 0)),
            pl.BlockSpec((blk, F), lambda i: (i, 0)),
            pl.BlockSpec((blk, F), lambda i: (i, 0)),
            pl.BlockSpec((blk, F), lambda i: (i, 0)),
            pl.BlockSpec((blk, F), lambda i: (i, 0)),
            pl.BlockSpec((blk, F), lambda i: (i, 0)),
            pl.BlockSpec((blk, 1), lambda i: (i, 0)),
            pl.BlockSpec((1, 1), lambda i: (0, 0)),
            pl.BlockSpec((13 * F, F), lambda i: (0, 0)),
            pl.BlockSpec((1, F), lambda i: (0, 0)),
            pl.BlockSpec((F, F), lambda i: (0, 0)),
            pl.BlockSpec((1, F), lambda i: (0, 0)),
        ],
        out_specs=pl.BlockSpec((blk, F), lambda i: (i, 0)),
        out_shape=jax.ShapeDtypeStruct((NP, F), jnp.float32),
    )(h, A, S1, S2, MN, MX, d, avg_log, pw, pb.reshape(1, F), lw,
      lb.reshape(1, F))


def _k3_body(h_ref, b_ref, fcw_ref, fcb_ref, fw_ref, fb_ref, out_ref, g_acc):
    i = pl.program_id(0)
    @pl.when(i == 0)
    def _():
        g_acc[...] = jnp.zeros_like(g_acc)
    bt = b_ref[...]
    oh = (bt == lax.broadcasted_iota(jnp.int32, (bt.shape[0], G), 1)
          ).astype(jnp.float32)
    g_acc[...] += lax.dot_general(oh, h_ref[...], (((0,), (0,)), ((), ())),
                                  preferred_element_type=jnp.float32)
    @pl.when(i == pl.num_programs(0) - 1)
    def _():
        g = g_acc[...] @ fcw_ref[...] + fcb_ref[...]
        out_ref[...] = jnp.dot(g, fw_ref[...],
                               preferred_element_type=jnp.float32) + fb_ref[...]


def _k3(h, batch, fcw, fcb, fw, fb):
    blk = 1024
    grid = NP // blk
    return pl.pallas_call(
        _k3_body,
        grid=(grid,),
        in_specs=[
            pl.BlockSpec((blk, F), lambda i: (i, 0)),
            pl.BlockSpec((blk, 1), lambda i: (i, 0)),
            pl.BlockSpec((F, F), lambda i: (0, 0)),
            pl.BlockSpec((1, F), lambda i: (0, 0)),
            pl.BlockSpec((F, OUT), lambda i: (0, 0)),
            pl.BlockSpec((1, OUT), lambda i: (0, 0)),
        ],
        out_specs=pl.BlockSpec((G, OUT), lambda i: (0, 0)),
        out_shape=jax.ShapeDtypeStruct((G, OUT), jnp.float32),
        scratch_shapes=[pltpu.VMEM((G, F), jnp.float32)],
    )(h, batch, fcw, fcb.reshape(1, F), fw, fb.reshape(1, OUT))


def kernel(x, edge_index, batch, pre_W, pre_b, post_W, post_b, lin_W, lin_b,
           fc_W, fc_b, f_W, f_b):
    xp = jnp.pad(x, ((0, NP - N), (0, 0)))
    batchp = jnp.pad(batch, (0, NP - N), constant_values=G).reshape(NP, 1)

    _edges_arr, edges_srt, bmeta, offtab = _sc_partition(
        edge_index[0], edge_index[1])

    h = xp
    d = None
    avg_log = None
    A, B = _k1(h, pre_W[0], pre_b[0])
    for l in range(L):
        S1, S2, MN, MX, DC = _sc_aggregate(B, edges_srt, bmeta, offtab)
        if l == 0:
            d = DC[:, :1]
            avg_log = _k0(d)
        if l < L - 1:
            h, A, B = _k2f(h, A, S1, S2, MN, MX, d, avg_log,
                           post_W[l], post_b[l], lin_W[l], lin_b[l],
                           pre_W[l + 1], pre_b[l + 1])
        else:
            h = _k2(h, A, S1, S2, MN, MX, d, avg_log,
                    post_W[l], post_b[l], lin_W[l], lin_b[l], relu=False)
    return _k3(h, batchp, fc_W, fc_b, f_W, f_b)
